# baseline (speedup 1.0000x reference)
import jax
import jax.numpy as jnp
from jax import lax
from jax.experimental import pallas as pl
from jax.experimental.pallas import tpu as pltpu
from jax.experimental.pallas import tpu_sc as plsc

L = 16
NC = 2
NS = 16
NW = NC * NS

B, H, W = 64, 512, 512
P = H * W
N = B * P
E = N // NW
K = 8192
CPP = P // K
NCHUNK = E // K
PAL = 4096


def _sc_body(x_hbm, pal_hbm, out_hbm, palv, pal0, pal1, pal2, xv, o0, o1, o2,
             isem0, isem1, osem0, osem1):
    c = lax.axis_index("c")
    s = lax.axis_index("s")
    wid = s * NC + c

    pltpu.sync_copy(pal_hbm.at[pl.ds(0, 3 * PAL)], palv)
    lane3 = lax.iota(jnp.int32, L) * 3

    @pl.loop(0, PAL // L)
    def _(k):
        idx = lane3 + k * (3 * L)
        pal0[pl.ds(k * L, L)] = plsc.load_gather(palv, [idx])
        pal1[pl.ds(k * L, L)] = plsc.load_gather(palv, [idx + 1])
        pal2[pl.ds(k * L, L)] = plsc.load_gather(palv, [idx + 2])

    isems = (isem0, isem1)
    osems = (osem0, osem1)

    def in_start(n, p):
        g = (wid * 2 + (n >> 5)) * P + (n & (CPP - 1)) * K
        pltpu.async_copy(x_hbm.at[pl.ds(g, K)], xv.at[p], isems[p])

    def in_wait(p):
        pltpu.make_async_copy(x_hbm.at[pl.ds(0, K)], xv.at[p], isems[p]).wait()

    def out_start(n, p):
        b = wid * 2 + (n >> 5)
        ob = b * 3 * P + (n & (CPP - 1)) * K
        pltpu.async_copy(o0.at[p], out_hbm.at[pl.ds(ob, K)], osems[p])
        pltpu.async_copy(o1.at[p], out_hbm.at[pl.ds(ob + P, K)], osems[p])
        pltpu.async_copy(o2.at[p], out_hbm.at[pl.ds(ob + 2 * P, K)], osems[p])

    def out_wait(p):
        for o in (o0, o1, o2):
            pltpu.make_async_copy(o.at[p], out_hbm.at[pl.ds(0, K)],
                                  osems[p]).wait()

    def compute(p):
        @plsc.parallel_loop(0, K // L, 1, unroll=8)
        def _(j):
            xx = xv[p, pl.ds(j * L, L)]
            t = xx * float(PAL)
            u = t + 0.5
            i = u.astype(jnp.int32)
            fi = i.astype(jnp.float32)
            is_half = (fi - t) == 0.5
            odd = (i & 1) == 1
            i = jnp.where(is_half & odd, i - 1, i)
            i = jnp.minimum(jnp.maximum(i, 0), PAL - 1)
            o0[p, pl.ds(j * L, L)] = plsc.load_gather(pal0, [i])
            o1[p, pl.ds(j * L, L)] = plsc.load_gather(pal1, [i])
            o2[p, pl.ds(j * L, L)] = plsc.load_gather(pal2, [i])

    in_start(0, 0)
    in_start(1, 1)

    @pl.loop(0, NCHUNK, step=2)
    def _(n2):
        for p in (0, 1):
            in_wait(p)

            @pl.when(n2 >= 2)
            def _():
                out_wait(p)

            compute(p)
            out_start(n2 + p, p)

            @pl.when(n2 < NCHUNK - 2)
            def _():
                in_start(n2 + p + 2, p)

    out_wait(0)
    out_wait(1)


@jax.jit
def _colormap_sc(xf, pal_t):
    mesh = plsc.VectorSubcoreMesh(core_axis_name="c", subcore_axis_name="s")
    f = pl.kernel(
        _sc_body,
        out_type=jax.ShapeDtypeStruct((B * 3 * P,), jnp.float32),
        mesh=mesh,
        compiler_params=pltpu.CompilerParams(needs_layout_passes=False),
        scratch_types=[
            pltpu.VMEM((3 * PAL,), jnp.float32),
            pltpu.VMEM((PAL,), jnp.float32),
            pltpu.VMEM((PAL,), jnp.float32),
            pltpu.VMEM((PAL,), jnp.float32),
            pltpu.VMEM((2, K), jnp.float32),
            pltpu.VMEM((2, K), jnp.float32),
            pltpu.VMEM((2, K), jnp.float32),
            pltpu.VMEM((2, K), jnp.float32),
            pltpu.SemaphoreType.DMA,
            pltpu.SemaphoreType.DMA,
            pltpu.SemaphoreType.DMA,
            pltpu.SemaphoreType.DMA,
        ],
    )
    return f(xf, pal_t)


def kernel(x, palette):
    xf = x.reshape(-1)
    pal_flat = palette.reshape(3 * PAL)
    out = _colormap_sc(xf, pal_flat)
    return out.reshape(B, 3, H, W)


# speedup vs baseline: 3.2251x; 3.2251x over previous
import jax
import jax.numpy as jnp
from jax import lax
from jax.experimental import pallas as pl
from jax.experimental.pallas import tpu as pltpu
from jax.experimental.pallas import tpu_sc as plsc

L = 16
NC = 2
NS = 16
NW = NC * NS

B, H, W = 64, 512, 512
R = 16
K = R * W
CPP = H // R
NCHUNK = 2 * CPP
PAL = 4096


def _sc_body(x_hbm, pal_hbm, out_hbm, palv, pal0, pal1, pal2, xv, o0, o1, o2,
             isem0, isem1, osem0, osem1):
    c = lax.axis_index("c")
    s = lax.axis_index("s")
    wid = s * NC + c

    pltpu.sync_copy(pal_hbm.at[pl.ds(0, 3 * PAL)], palv)
    lane3 = lax.iota(jnp.int32, L) * 3

    @pl.loop(0, PAL // L)
    def _(k):
        idx = lane3 + k * (3 * L)
        pal0[pl.ds(k * L, L)] = plsc.load_gather(palv, [idx])
        pal1[pl.ds(k * L, L)] = plsc.load_gather(palv, [idx + 1])
        pal2[pl.ds(k * L, L)] = plsc.load_gather(palv, [idx + 2])

    isems = (isem0, isem1)
    osems = (osem0, osem1)

    def in_start(n, p):
        b = wid * 2 + (n >> 5)
        r = (n & (CPP - 1)) * R
        pltpu.async_copy(x_hbm.at[b, pl.ds(r, R), :], xv.at[p], isems[p])

    def in_wait(p):
        pltpu.make_async_copy(x_hbm.at[0, pl.ds(0, R), :], xv.at[p],
                              isems[p]).wait()

    def out_start(n, p):
        b = wid * 2 + (n >> 5)
        r = (n & (CPP - 1)) * R
        pltpu.async_copy(o0.at[p], out_hbm.at[b, 0, pl.ds(r, R), :], osems[p])
        pltpu.async_copy(o1.at[p], out_hbm.at[b, 1, pl.ds(r, R), :], osems[p])
        pltpu.async_copy(o2.at[p], out_hbm.at[b, 2, pl.ds(r, R), :], osems[p])

    def out_wait(p):
        for o in (o0, o1, o2):
            pltpu.make_async_copy(o.at[p], out_hbm.at[0, 0, pl.ds(0, R), :],
                                  osems[p]).wait()

    def compute(p):
        @plsc.parallel_loop(0, K // L, 1, unroll=8)
        def _(j):
            row = j >> 5
            col = (j & 31) * L
            xx = xv[p, row, pl.ds(col, L)]
            t = xx * float(PAL)
            u = t + 0.5
            i = u.astype(jnp.int32)
            fi = i.astype(jnp.float32)
            is_half = (fi - t) == 0.5
            odd = (i & 1) == 1
            i = jnp.where(is_half & odd, i - 1, i)
            i = jnp.minimum(jnp.maximum(i, 0), PAL - 1)
            o0[p, row, pl.ds(col, L)] = plsc.load_gather(pal0, [i])
            o1[p, row, pl.ds(col, L)] = plsc.load_gather(pal1, [i])
            o2[p, row, pl.ds(col, L)] = plsc.load_gather(pal2, [i])

    in_start(0, 0)
    in_start(1, 1)

    @pl.loop(0, NCHUNK, step=2)
    def _(n2):
        for p in (0, 1):
            in_wait(p)

            @pl.when(n2 >= 2)
            def _():
                out_wait(p)

            compute(p)
            out_start(n2 + p, p)

            @pl.when(n2 < NCHUNK - 2)
            def _():
                in_start(n2 + p + 2, p)

    out_wait(0)
    out_wait(1)


@jax.jit
def _colormap_sc(x, pal_flat):
    mesh = plsc.VectorSubcoreMesh(core_axis_name="c", subcore_axis_name="s")
    f = pl.kernel(
        _sc_body,
        out_type=jax.ShapeDtypeStruct((B, 3, H, W), jnp.float32),
        mesh=mesh,
        compiler_params=pltpu.CompilerParams(needs_layout_passes=False),
        scratch_types=[
            pltpu.VMEM((3 * PAL,), jnp.float32),
            pltpu.VMEM((PAL,), jnp.float32),
            pltpu.VMEM((PAL,), jnp.float32),
            pltpu.VMEM((PAL,), jnp.float32),
            pltpu.VMEM((2, R, W), jnp.float32),
            pltpu.VMEM((2, R, W), jnp.float32),
            pltpu.VMEM((2, R, W), jnp.float32),
            pltpu.VMEM((2, R, W), jnp.float32),
            pltpu.SemaphoreType.DMA,
            pltpu.SemaphoreType.DMA,
            pltpu.SemaphoreType.DMA,
            pltpu.SemaphoreType.DMA,
        ],
    )
    return f(x, pal_flat)


def kernel(x, palette):
    pal_flat = palette.reshape(3 * PAL)
    return _colormap_sc(x, pal_flat)
